# Initial kernel scaffold; baseline (speedup 1.0000x reference)
import jax
import jax.numpy as jnp
from jax.experimental import pallas as pl


def kernel(a, d, gate_Wt, gate_bt, exp_Wt, exp_bt, gate_Wd, gate_bd, exp_Wd, exp_bd, conv_W, conv_b):
    raise NotImplementedError("write your pallas kernel here")


# speedup vs baseline: 2.2101x; 2.2101x over previous
import functools

import jax
import jax.numpy as jnp
from jax.experimental import pallas as pl
from jax.experimental.pallas import tpu as pltpu

B, L, DIN, DOUT, E = 4, 2048, 64, 128, 8
C = 2 * DOUT
K = 5
PAD = K // 2
CH = 512
NCH = L // CH


def _moe_chunk(x, gw, gb, ew, eb):
    logits = jnp.dot(x, gw, preferred_element_type=jnp.float32) + gb
    m = jnp.max(logits, axis=-1, keepdims=True)
    p = jnp.exp(logits - m)
    gates = p / jnp.sum(p, axis=-1, keepdims=True)
    h = jnp.dot(x, ew, preferred_element_type=jnp.float32) + eb
    h = h * jax.nn.sigmoid(h)
    acc = gates[:, 0:1] * h[:, 0:DOUT]
    for e in range(1, E):
        acc += gates[:, e:e + 1] * h[:, e * DOUT:(e + 1) * DOUT]
    return acc


def _body(a_ref, d_ref, gwt_ref, gbt_ref, ewt_ref, ebt_ref,
          gwd_ref, gbd_ref, ewd_ref, ebd_ref, wk_ref, cb_ref,
          out_ref, comb_ref):
    comb_ref[0:PAD, :] = jnp.zeros((PAD, C), jnp.float32)
    comb_ref[pl.ds(L + PAD, PAD), :] = jnp.zeros((PAD, C), jnp.float32)

    for c in range(NCH):
        xa = a_ref[0, pl.ds(c * CH, CH), :]
        xd = d_ref[0, pl.ds(c * CH, CH), :]
        fa = _moe_chunk(xa, gwt_ref[...], gbt_ref[...], ewt_ref[...], ebt_ref[...])
        fd = _moe_chunk(xd, gwd_ref[...], gbd_ref[...], ewd_ref[...], ebd_ref[...])
        comb_ref[pl.ds(PAD + c * CH, CH), 0:DOUT] = fa
        comb_ref[pl.ds(PAD + c * CH, CH), DOUT:C] = fd

    cb = cb_ref[...]
    for c in range(NCH):
        y = jnp.dot(comb_ref[pl.ds(c * CH, CH), :], wk_ref[0],
                    preferred_element_type=jnp.float32)
        for k in range(1, K):
            y += jnp.dot(comb_ref[pl.ds(c * CH + k, CH), :], wk_ref[k],
                         preferred_element_type=jnp.float32)
        out_ref[0, pl.ds(c * CH, CH), :] = jnp.maximum(y + cb, 0.0)


@jax.jit
def kernel(a, d, gate_Wt, gate_bt, exp_Wt, exp_bt,
           gate_Wd, gate_bd, exp_Wd, exp_bd, conv_W, conv_b):
    ewt = jnp.transpose(exp_Wt, (1, 0, 2)).reshape(DIN, E * DOUT)
    ewd = jnp.transpose(exp_Wd, (1, 0, 2)).reshape(DIN, E * DOUT)
    ebt = exp_bt.reshape(1, E * DOUT)
    ebd = exp_bd.reshape(1, E * DOUT)
    wk = jnp.transpose(conv_W, (2, 1, 0))
    gbt = gate_bt.reshape(1, E)
    gbd = gate_bd.reshape(1, E)
    cb = conv_b.reshape(1, C)

    full = lambda shape: pl.BlockSpec(shape, lambda b: (0,) * len(shape))
    return pl.pallas_call(
        _body,
        grid=(B,),
        in_specs=[
            pl.BlockSpec((1, L, DIN), lambda b: (b, 0, 0)),
            pl.BlockSpec((1, L, DIN), lambda b: (b, 0, 0)),
            full((DIN, E)), full((1, E)), full((DIN, E * DOUT)), full((1, E * DOUT)),
            full((DIN, E)), full((1, E)), full((DIN, E * DOUT)), full((1, E * DOUT)),
            full((K, C, C)), full((1, C)),
        ],
        out_specs=pl.BlockSpec((1, L, C), lambda b: (b, 0, 0)),
        out_shape=jax.ShapeDtypeStruct((B, L, C), jnp.float32),
        scratch_shapes=[pltpu.VMEM((L + 2 * PAD, C), jnp.float32)],
    )(a, d, gate_Wt, gbt, ewt, ebt, gate_Wd, gbd, ewd, ebd, wk, cb)


# speedup vs baseline: 2.2109x; 1.0004x over previous
import functools

import jax
import jax.numpy as jnp
from jax.experimental import pallas as pl
from jax.experimental.pallas import tpu as pltpu

B, L, DIN, DOUT, E = 4, 2048, 64, 128, 8
C = 2 * DOUT
K = 5
PAD = K // 2
CH = 512
NCH = L // CH


def _moe_chunk(x, gw, gb, ew, eb):
    logits = jnp.dot(x, gw, preferred_element_type=jnp.float32) + gb
    m = jnp.max(logits, axis=-1, keepdims=True)
    p = jnp.exp(logits - m)
    gates = p / jnp.sum(p, axis=-1, keepdims=True)
    h = jnp.dot(x, ew, preferred_element_type=jnp.float32) + eb
    h = h * jax.nn.sigmoid(h)
    acc = gates[:, 0:1] * h[:, 0:DOUT]
    for e in range(1, E):
        acc += gates[:, e:e + 1] * h[:, e * DOUT:(e + 1) * DOUT]
    return acc


def _body(a_ref, d_ref, gwt_ref, gbt_ref, ewt_ref, ebt_ref,
          gwd_ref, gbd_ref, ewd_ref, ebd_ref, wk_ref, cb_ref,
          out_ref, comb_ref):
    comb_ref[0:PAD, :] = jnp.zeros((PAD, C), jnp.bfloat16)
    comb_ref[pl.ds(L + PAD, PAD), :] = jnp.zeros((PAD, C), jnp.bfloat16)

    for c in range(NCH):
        xa = a_ref[0, pl.ds(c * CH, CH), :].astype(jnp.bfloat16)
        xd = d_ref[0, pl.ds(c * CH, CH), :].astype(jnp.bfloat16)
        fa = _moe_chunk(xa, gwt_ref[...], gbt_ref[...], ewt_ref[...], ebt_ref[...])
        fd = _moe_chunk(xd, gwd_ref[...], gbd_ref[...], ewd_ref[...], ebd_ref[...])
        comb_ref[pl.ds(PAD + c * CH, CH), 0:DOUT] = fa.astype(jnp.bfloat16)
        comb_ref[pl.ds(PAD + c * CH, CH), DOUT:C] = fd.astype(jnp.bfloat16)

    cb = cb_ref[...]
    for c in range(NCH):
        y = jnp.dot(comb_ref[pl.ds(c * CH, CH), :], wk_ref[0],
                    preferred_element_type=jnp.float32)
        for k in range(1, K):
            y += jnp.dot(comb_ref[pl.ds(c * CH + k, CH), :], wk_ref[k],
                         preferred_element_type=jnp.float32)
        out_ref[0, pl.ds(c * CH, CH), :] = jnp.maximum(y + cb, 0.0)


@jax.jit
def kernel(a, d, gate_Wt, gate_bt, exp_Wt, exp_bt,
           gate_Wd, gate_bd, exp_Wd, exp_bd, conv_W, conv_b):
    ewt = jnp.transpose(exp_Wt, (1, 0, 2)).reshape(DIN, E * DOUT).astype(jnp.bfloat16)
    ewd = jnp.transpose(exp_Wd, (1, 0, 2)).reshape(DIN, E * DOUT).astype(jnp.bfloat16)
    ebt = exp_bt.reshape(1, E * DOUT)
    ebd = exp_bd.reshape(1, E * DOUT)
    wk = jnp.transpose(conv_W, (2, 1, 0)).astype(jnp.bfloat16)
    gwt = gate_Wt.astype(jnp.bfloat16)
    gwd = gate_Wd.astype(jnp.bfloat16)
    gbt = gate_bt.reshape(1, E)
    gbd = gate_bd.reshape(1, E)
    cb = conv_b.reshape(1, C)

    full = lambda shape: pl.BlockSpec(shape, lambda b: (0,) * len(shape))
    return pl.pallas_call(
        _body,
        grid=(B,),
        in_specs=[
            pl.BlockSpec((1, L, DIN), lambda b: (b, 0, 0)),
            pl.BlockSpec((1, L, DIN), lambda b: (b, 0, 0)),
            full((DIN, E)), full((1, E)), full((DIN, E * DOUT)), full((1, E * DOUT)),
            full((DIN, E)), full((1, E)), full((DIN, E * DOUT)), full((1, E * DOUT)),
            full((K, C, C)), full((1, C)),
        ],
        out_specs=pl.BlockSpec((1, L, C), lambda b: (b, 0, 0)),
        out_shape=jax.ShapeDtypeStruct((B, L, C), jnp.float32),
        scratch_shapes=[pltpu.VMEM((L + 2 * PAD, C), jnp.bfloat16)],
    )(a, d, gwt, gbt, ewt, ebt, gwd, gbd, ewd, ebd, wk, cb)


# speedup vs baseline: 2.2158x; 1.0022x over previous
import functools

import jax
import jax.numpy as jnp
from jax.experimental import pallas as pl
from jax.experimental.pallas import tpu as pltpu

B, L, DIN, DOUT, E = 4, 2048, 64, 128, 8
C = 2 * DOUT
K = 5
PAD = K // 2
CH = 512
NCH = L // CH


def _moe_chunk(x, gw, gb, ew, eb):
    logits = jnp.dot(x, gw, preferred_element_type=jnp.float32) + gb
    m = jnp.max(logits, axis=-1, keepdims=True)
    p = jnp.exp(logits - m)
    gates = p / jnp.sum(p, axis=-1, keepdims=True)
    h = jnp.dot(x, ew, preferred_element_type=jnp.float32) + eb
    h = h * jax.nn.sigmoid(h)
    acc = gates[:, 0:1] * h[:, 0:DOUT]
    for e in range(1, E):
        acc += gates[:, e:e + 1] * h[:, e * DOUT:(e + 1) * DOUT]
    return acc


def _body(a_ref, d_ref, gwt_ref, gbt_ref, ewt_ref, ebt_ref,
          gwd_ref, gbd_ref, ewd_ref, ebd_ref, wk_ref, cb_ref,
          out_ref, comb_ref):
    comb_ref[0:PAD, :] = jnp.zeros((PAD, C), jnp.bfloat16)
    comb_ref[pl.ds(L + PAD, PAD), :] = jnp.zeros((PAD, C), jnp.bfloat16)

    for c in range(NCH):
        xa = a_ref[0, pl.ds(c * CH, CH), :].astype(jnp.bfloat16)
        xd = d_ref[0, pl.ds(c * CH, CH), :].astype(jnp.bfloat16)
        fa = _moe_chunk(xa, gwt_ref[...], gbt_ref[...], ewt_ref[...], ebt_ref[...])
        fd = _moe_chunk(xd, gwd_ref[...], gbd_ref[...], ewd_ref[...], ebd_ref[...])
        comb_ref[pl.ds(PAD + c * CH, CH), 0:DOUT] = fa.astype(jnp.bfloat16)
        comb_ref[pl.ds(PAD + c * CH, CH), DOUT:C] = fd.astype(jnp.bfloat16)

    cb = cb_ref[...]
    for c in range(NCH):
        y = jnp.dot(comb_ref[pl.ds(c * CH, CH), :], wk_ref[0],
                    preferred_element_type=jnp.float32)
        for k in range(1, K):
            y += jnp.dot(comb_ref[pl.ds(c * CH + k, CH), :], wk_ref[k],
                         preferred_element_type=jnp.float32)
        out_ref[0, pl.ds(c * CH, CH), :] = jnp.maximum(y + cb, 0.0)


@jax.jit
def kernel(a, d, gate_Wt, gate_bt, exp_Wt, exp_bt,
           gate_Wd, gate_bd, exp_Wd, exp_bd, conv_W, conv_b):
    ewt = jnp.transpose(exp_Wt, (1, 0, 2)).reshape(DIN, E * DOUT).astype(jnp.bfloat16)
    ewd = jnp.transpose(exp_Wd, (1, 0, 2)).reshape(DIN, E * DOUT).astype(jnp.bfloat16)
    ebt = exp_bt.reshape(1, E * DOUT)
    ebd = exp_bd.reshape(1, E * DOUT)
    wk = jnp.transpose(conv_W, (2, 1, 0)).astype(jnp.bfloat16)
    gwt = gate_Wt.astype(jnp.bfloat16)
    gwd = gate_Wd.astype(jnp.bfloat16)
    gbt = gate_bt.reshape(1, E)
    gbd = gate_bd.reshape(1, E)
    cb = conv_b.reshape(1, C)

    full = lambda shape: pl.BlockSpec(shape, lambda b: (0,) * len(shape))
    return pl.pallas_call(
        _body,
        grid=(B,),
        in_specs=[
            pl.BlockSpec((1, L, DIN), lambda b: (b, 0, 0)),
            pl.BlockSpec((1, L, DIN), lambda b: (b, 0, 0)),
            full((DIN, E)), full((1, E)), full((DIN, E * DOUT)), full((1, E * DOUT)),
            full((DIN, E)), full((1, E)), full((DIN, E * DOUT)), full((1, E * DOUT)),
            full((K, C, C)), full((1, C)),
        ],
        out_specs=pl.BlockSpec((1, L, C), lambda b: (b, 0, 0)),
        out_shape=jax.ShapeDtypeStruct((B, L, C), jnp.float32),
        compiler_params=pltpu.CompilerParams(dimension_semantics=("parallel",)),
        scratch_shapes=[pltpu.VMEM((L + 2 * PAD, C), jnp.bfloat16)],
    )(a, d, gwt, gbt, ewt, ebt, gwd, gbd, ewd, ebd, wk, cb)


# speedup vs baseline: 2.4201x; 1.0922x over previous
import functools

import jax
import jax.numpy as jnp
from jax import lax
from jax.experimental import pallas as pl
from jax.experimental.pallas import tpu as pltpu

B, L, DIN, DOUT, E = 4, 2048, 64, 128, 8
C = 2 * DOUT
K = 5
PAD = K // 2
CH = 512
NCH = L // CH
DA = DIN + 1


def _moe_chunk(xa, gw, ew):
    logits = jnp.dot(xa, gw, preferred_element_type=jnp.float32)
    m = jnp.max(logits, axis=-1, keepdims=True)
    p = jnp.exp(logits - m)
    gates = (p / jnp.sum(p, axis=-1, keepdims=True)).astype(jnp.bfloat16)
    u = jnp.dot(xa, ew, preferred_element_type=jnp.float32).astype(jnp.bfloat16)
    q = u + u * jnp.tanh(u)
    acc = gates[:, 0:1] * q[:, 0:DOUT]
    for e in range(1, E):
        acc += gates[:, e:e + 1] * q[:, e * DOUT:(e + 1) * DOUT]
    return acc


def _body(a_ref, d_ref, gwt_ref, ewt_ref, gwd_ref, ewd_ref,
          wk_ref, cb_ref, out_ref):
    gwt, ewt = gwt_ref[...], ewt_ref[...]
    gwd, ewd = gwd_ref[...], ewd_ref[...]
    cb = cb_ref[...]
    zpad = jnp.zeros((PAD, C), jnp.bfloat16)

    for c in range(NCH):
        lo = max(0, c * CH - PAD)
        hi = min(L, (c + 1) * CH + PAD)
        n = hi - lo
        ones = jnp.ones((n, 1), jnp.bfloat16)
        xa = jnp.concatenate(
            [a_ref[0, pl.ds(lo, n), :].astype(jnp.bfloat16), ones], axis=1)
        xd = jnp.concatenate(
            [d_ref[0, pl.ds(lo, n), :].astype(jnp.bfloat16), ones], axis=1)
        fa = _moe_chunk(xa, gwt, ewt)
        fd = _moe_chunk(xd, gwd, ewd)
        comb = jnp.concatenate([fa, fd], axis=1)
        if lo == 0:
            comb = jnp.concatenate([zpad, comb], axis=0)
        if hi == L:
            comb = jnp.concatenate([comb, zpad], axis=0)
        y = jnp.dot(lax.slice(comb, (0, 0), (CH, C)), wk_ref[0],
                    preferred_element_type=jnp.float32)
        for k in range(1, K):
            y += jnp.dot(lax.slice(comb, (k, 0), (k + CH, C)), wk_ref[k],
                         preferred_element_type=jnp.float32)
        out_ref[0, pl.ds(c * CH, CH), :] = jnp.maximum(y + cb, 0.0)


@jax.jit
def kernel(a, d, gate_Wt, gate_bt, exp_Wt, exp_bt,
           gate_Wd, gate_bd, exp_Wd, exp_bd, conv_W, conv_b):
    def aug_expert(ew, eb):
        w = jnp.transpose(ew, (1, 0, 2)).reshape(DIN, E * DOUT)
        return jnp.concatenate([0.5 * w, 0.5 * eb.reshape(1, E * DOUT)],
                               axis=0).astype(jnp.bfloat16)

    def aug_gate(gw, gb):
        return jnp.concatenate([gw, gb.reshape(1, E)], axis=0).astype(jnp.bfloat16)

    ewt = aug_expert(exp_Wt, exp_bt)
    ewd = aug_expert(exp_Wd, exp_bd)
    gwt = aug_gate(gate_Wt, gate_bt)
    gwd = aug_gate(gate_Wd, gate_bd)
    wk = jnp.transpose(conv_W, (2, 1, 0)).astype(jnp.bfloat16)
    cb = conv_b.reshape(1, C)

    full = lambda shape: pl.BlockSpec(shape, lambda b: (0,) * len(shape))
    return pl.pallas_call(
        _body,
        grid=(B,),
        in_specs=[
            pl.BlockSpec((1, L, DIN), lambda b: (b, 0, 0)),
            pl.BlockSpec((1, L, DIN), lambda b: (b, 0, 0)),
            full((DA, E)), full((DA, E * DOUT)),
            full((DA, E)), full((DA, E * DOUT)),
            full((K, C, C)), full((1, C)),
        ],
        out_specs=pl.BlockSpec((1, L, C), lambda b: (b, 0, 0)),
        out_shape=jax.ShapeDtypeStruct((B, L, C), jnp.float32),
        compiler_params=pltpu.CompilerParams(dimension_semantics=("parallel",)),
    )(a, d, gwt, ewt, gwd, ewd, wk, cb)


# speedup vs baseline: 2.5766x; 1.0647x over previous
import jax
import jax.numpy as jnp
from jax import lax
from jax.experimental import pallas as pl
from jax.experimental.pallas import tpu as pltpu

B, L, DIN, DOUT, E = 4, 2048, 64, 128, 8
C = 2 * DOUT
K = 5
PAD = K // 2
CH = 512
NCH = L // CH
DA = DIN + 1


def _moe_chunk(xa, gw, ew):
    logits = jnp.dot(xa, gw, preferred_element_type=jnp.float32)
    m = jnp.max(logits, axis=-1, keepdims=True)
    p = jnp.exp(logits - m)
    gates = (p / jnp.sum(p, axis=-1, keepdims=True)).astype(jnp.bfloat16)
    u = jnp.dot(xa, ew, preferred_element_type=jnp.float32).astype(jnp.bfloat16)
    q = u + u * jnp.tanh(u)
    acc = gates[:, 0:1] * q[:, 0:DOUT]
    for e in range(1, E):
        acc += gates[:, e:e + 1] * q[:, e * DOUT:(e + 1) * DOUT]
    return acc


def _body(a_ref, d_ref, gwt_ref, gbt_ref, ewt_ref, ebt_ref,
          gwd_ref, gbd_ref, ewd_ref, ebd_ref, wk_ref, cb_ref,
          out_ref, gwt_s, ewt_s, gwd_s, ewd_s):
    @pl.when(pl.program_id(0) == 0)
    def _init():
        for gs, gref, gbref, es, eref, ebref in (
                (gwt_s, gwt_ref, gbt_ref, ewt_s, ewt_ref, ebt_ref),
                (gwd_s, gwd_ref, gbd_ref, ewd_s, ewd_ref, ebd_ref)):
            gs[0:DIN, :] = gref[...].astype(jnp.bfloat16)
            gs[DIN:DA, :] = gbref[...].astype(jnp.bfloat16)
            for e in range(E):
                sl = slice(e * DOUT, (e + 1) * DOUT)
                es[0:DIN, sl] = (0.5 * eref[e]).astype(jnp.bfloat16)
                es[DIN:DA, sl] = (0.5 * ebref[e:e + 1, :]).astype(jnp.bfloat16)

    gwt, ewt = gwt_s[...], ewt_s[...]
    gwd, ewd = gwd_s[...], ewd_s[...]
    cb = cb_ref[...]
    zpad = jnp.zeros((PAD, C), jnp.bfloat16)

    for c in range(NCH):
        lo = max(0, c * CH - PAD)
        hi = min(L, (c + 1) * CH + PAD)
        n = hi - lo
        ones = jnp.ones((n, 1), jnp.bfloat16)
        xa = jnp.concatenate(
            [a_ref[0, pl.ds(lo, n), :].astype(jnp.bfloat16), ones], axis=1)
        xd = jnp.concatenate(
            [d_ref[0, pl.ds(lo, n), :].astype(jnp.bfloat16), ones], axis=1)
        fa = _moe_chunk(xa, gwt, ewt)
        fd = _moe_chunk(xd, gwd, ewd)
        comb = jnp.concatenate([fa, fd], axis=1)
        if lo == 0:
            comb = jnp.concatenate([zpad, comb], axis=0)
        if hi == L:
            comb = jnp.concatenate([comb, zpad], axis=0)
        y = jnp.dot(lax.slice(comb, (0, 0), (CH, C)), wk_ref[0],
                    preferred_element_type=jnp.float32)
        for k in range(1, K):
            y += jnp.dot(lax.slice(comb, (k, 0), (k + CH, C)), wk_ref[k],
                         preferred_element_type=jnp.float32)
        out_ref[0, pl.ds(c * CH, CH), :] = jnp.maximum(y + cb, 0.0)


@jax.jit
def kernel(a, d, gate_Wt, gate_bt, exp_Wt, exp_bt,
           gate_Wd, gate_bd, exp_Wd, exp_bd, conv_W, conv_b):
    wk = jnp.transpose(conv_W, (2, 1, 0)).astype(jnp.bfloat16)
    gbt = gate_bt.reshape(1, E)
    gbd = gate_bd.reshape(1, E)
    cb = conv_b.reshape(1, C)

    full = lambda shape: pl.BlockSpec(shape, lambda b: (0,) * len(shape))
    return pl.pallas_call(
        _body,
        grid=(B,),
        in_specs=[
            pl.BlockSpec((1, L, DIN), lambda b: (b, 0, 0)),
            pl.BlockSpec((1, L, DIN), lambda b: (b, 0, 0)),
            full((DIN, E)), full((1, E)), full((E, DIN, DOUT)), full((E, DOUT)),
            full((DIN, E)), full((1, E)), full((E, DIN, DOUT)), full((E, DOUT)),
            full((K, C, C)), full((1, C)),
        ],
        out_specs=pl.BlockSpec((1, L, C), lambda b: (b, 0, 0)),
        out_shape=jax.ShapeDtypeStruct((B, L, C), jnp.float32),
        compiler_params=pltpu.CompilerParams(dimension_semantics=("arbitrary",)),
        scratch_shapes=[
            pltpu.VMEM((DA, E), jnp.bfloat16),
            pltpu.VMEM((DA, E * DOUT), jnp.bfloat16),
            pltpu.VMEM((DA, E), jnp.bfloat16),
            pltpu.VMEM((DA, E * DOUT), jnp.bfloat16),
        ],
    )(a, d, gate_Wt, gbt, exp_Wt, exp_bt, gate_Wd, gbd, exp_Wd, exp_bd, wk, cb)
